# baseline (speedup 1.0000x reference)
import functools

import jax
import jax.numpy as jnp
from jax import lax
from jax.experimental import pallas as pl
from jax.experimental.pallas import tpu as pltpu
from jax.experimental.pallas import tpu_sc as plsc

_NC = 2
_NS = 16
_NW = _NC * _NS
_LANES = 16
_CHUNK = 128

_BS = 16


def _combined_table(speaker_table, position_table, token_type_table, L):
    P, H = speaker_table.shape

    def body(spk_ref, pos_ref, tt_ref, out_ref):
        spk = spk_ref[...]
        postt = pos_ref[...] + tt_ref[...]
        out_ref[...] = spk[:, None, :] + postt[None, :, :]

    out = pl.pallas_call(
        body,
        grid=(P // _BS,),
        in_specs=[
            pl.BlockSpec((_BS, H), lambda i: (i, 0)),
            pl.BlockSpec((L, H), lambda i: (0, 0)),
            pl.BlockSpec((1, H), lambda i: (0, 0)),
        ],
        out_specs=pl.BlockSpec((_BS, L, H), lambda i: (i, 0, 0)),
        out_shape=jax.ShapeDtypeStruct((P, L, H), jnp.float32),
    )(speaker_table, position_table[:L], token_type_table[0:1])
    return out.reshape(P * L, H)


def _build_sc_kernel(N, H):
    assert N % (_NW * _CHUNK) == 0
    rows_per_w = N // _NW
    n_chunks = rows_per_w // _CHUNK

    mesh = plsc.VectorSubcoreMesh(core_axis_name="c", subcore_axis_name="s")

    @functools.partial(
        pl.kernel,
        out_type=jax.ShapeDtypeStruct((N, H), jnp.float32),
        mesh=mesh,
        scratch_types=[
            pltpu.VMEM((2, 2, _CHUNK), jnp.int32),
            pltpu.VMEM((3, _CHUNK, H), jnp.float32),
            pltpu.VMEM((2, _CHUNK, H), jnp.float32),
            pltpu.SemaphoreType.DMA,
            pltpu.SemaphoreType.DMA,
            pltpu.SemaphoreType.DMA,
            pltpu.SemaphoreType.DMA,
        ],
    )
    def sc_embed(ids_hbm, word_hbm, comb_hbm,
                 out_hbm, idx_v, obuf, cbuf, sem_i, sem_w, sem_c, sem_o):
        wid = lax.axis_index("s") * _NC + lax.axis_index("c")
        wbase = wid * rows_per_w

        def fetch_ids(ck):
            s2 = lax.rem(ck, 2)
            rowbase = wbase + ck * _CHUNK
            pltpu.async_copy(ids_hbm.at[:, pl.ds(rowbase, _CHUNK)],
                             idx_v.at[s2], sem_i)

        def wait_ids():
            pltpu.make_async_copy(ids_hbm.at[:, pl.ds(0, _CHUNK)],
                                  idx_v.at[0], sem_i).wait()

        def issue(ck):
            s3 = lax.rem(ck, 3)
            s2 = lax.rem(ck, 2)
            pltpu.async_copy(word_hbm.at[idx_v.at[s2, 0]], obuf.at[s3],
                             sem_w)
            pltpu.async_copy(comb_hbm.at[idx_v.at[s2, 1]], cbuf.at[s2],
                             sem_c)

        fetch_ids(0)
        wait_ids()
        issue(0)
        fetch_ids(1)

        @pl.loop(0, n_chunks)
        def _(ck):
            s3 = lax.rem(ck, 3)
            s2 = lax.rem(ck, 2)
            rowbase = wbase + ck * _CHUNK

            pltpu.make_async_copy(word_hbm.at[idx_v.at[s2, 0]], obuf.at[s3],
                                  sem_w).wait()
            pltpu.make_async_copy(comb_hbm.at[idx_v.at[s2, 1]], cbuf.at[s2],
                                  sem_c).wait()

            @pl.when(ck + 2 < n_chunks)
            def _():
                fetch_ids(ck + 2)

            @pl.when(ck + 1 < n_chunks)
            def _():
                wait_ids()

                @pl.when(ck >= 2)
                def _():
                    pltpu.make_async_copy(
                        obuf.at[0], out_hbm.at[pl.ds(0, _CHUNK)],
                        sem_o).wait()

                issue(ck + 1)

            @plsc.parallel_loop(0, _CHUNK, unroll=4)
            def _(r):
                for c in range(H // _LANES):
                    sl = pl.ds(c * _LANES, _LANES)
                    plsc.addupdate(obuf.at[s3, r, sl], cbuf[s2, r, sl])

            pltpu.async_copy(obuf.at[s3],
                             out_hbm.at[pl.ds(rowbase, _CHUNK)], sem_o)

        for _ in range(3):
            pltpu.make_async_copy(obuf.at[0], out_hbm.at[pl.ds(0, _CHUNK)],
                                  sem_o).wait()

    return sc_embed


def kernel(input_ids, speaker_ids, word_table, position_table,
           token_type_table, speaker_table):
    B, L = input_ids.shape
    V, H = word_table.shape
    N = B * L
    comb = _combined_table(speaker_table, position_table,
                           token_type_table, L)
    comb_idx = (speaker_ids.astype(jnp.int32) * L
                + jnp.arange(L, dtype=jnp.int32)[None, :])
    ids = jnp.stack([input_ids.reshape(N).astype(jnp.int32),
                     comb_idx.reshape(N)])
    sc = _build_sc_kernel(N, H)
    out = sc(ids, word_table, comb)
    return out.reshape(B, L, H)


# speedup vs baseline: 1.2249x; 1.2249x over previous
import functools

import jax
import jax.numpy as jnp
from jax import lax
from jax.experimental import pallas as pl
from jax.experimental.pallas import tpu as pltpu
from jax.experimental.pallas import tpu_sc as plsc

_NC = 2
_NS = 16
_NW = _NC * _NS
_LANES = 16
_CHUNK = 128


def _build_sc_kernel(N, L, H, P):
    assert N % (_NW * _CHUNK) == 0
    rows_per_w = N // _NW
    n_chunks = rows_per_w // _CHUNK

    mesh = plsc.VectorSubcoreMesh(core_axis_name="c", subcore_axis_name="s")

    @functools.partial(
        pl.kernel,
        out_type=jax.ShapeDtypeStruct((N, H), jnp.float32),
        mesh=mesh,
        scratch_types=[
            pltpu.VMEM((2, 2, _CHUNK), jnp.int32),
            pltpu.VMEM((3, _CHUNK, H), jnp.float32),
            pltpu.VMEM((L, H), jnp.float32),
            pltpu.VMEM((H,), jnp.float32),
            pltpu.VMEM_SHARED((P, H), jnp.float32),
            pltpu.SemaphoreType.DMA,
            pltpu.SemaphoreType.DMA,
            pltpu.SemaphoreType.DMA,
            pltpu.SemaphoreType.DMA,
        ],
    )
    def sc_embed(ids_hbm, word_hbm, pos_hbm, tt_hbm, spk_hbm,
                 out_hbm, idx_v, obuf, posbuf, ttbuf,
                 spk_sp, sem_i, sem_w, sem_s, sem_o):
        wid = lax.axis_index("s") * _NC + lax.axis_index("c")
        wbase = wid * rows_per_w

        @pl.when(lax.axis_index("s") == 0)
        def _():
            pltpu.sync_copy(spk_hbm, spk_sp)

        pltpu.sync_copy(pos_hbm.at[pl.ds(0, L)], posbuf)
        pltpu.sync_copy(tt_hbm.at[0], ttbuf)

        @plsc.parallel_loop(0, L, unroll=2)
        def _(r):
            for c in range(H // _LANES):
                sl = pl.ds(c * _LANES, _LANES)
                posbuf[r, sl] = posbuf[r, sl] + ttbuf[sl]

        plsc.subcore_barrier()

        def fetch_ids(ck):
            s2 = lax.rem(ck, 2)
            rowbase = wbase + ck * _CHUNK
            pltpu.async_copy(ids_hbm.at[:, pl.ds(rowbase, _CHUNK)],
                             idx_v.at[s2], sem_i)

        def wait_ids():
            pltpu.make_async_copy(ids_hbm.at[:, pl.ds(0, _CHUNK)],
                                  idx_v.at[0], sem_i).wait()

        def issue_word(ck):
            s3 = lax.rem(ck, 3)
            s2 = lax.rem(ck, 2)
            pltpu.async_copy(word_hbm.at[idx_v.at[s2, 0]], obuf.at[s3],
                             sem_w)

        def wait_word():
            pltpu.make_async_copy(word_hbm.at[idx_v.at[0, 0]], obuf.at[0],
                                  sem_w).wait()

        def issue_spk_add(ck):
            s3 = lax.rem(ck, 3)
            s2 = lax.rem(ck, 2)
            pltpu.async_copy(spk_sp.at[idx_v.at[s2, 1]], obuf.at[s3],
                             sem_s, add=True)

        def wait_spk_add():
            pltpu.make_async_copy(spk_sp.at[idx_v.at[0, 1]], obuf.at[0],
                                  sem_s).wait()

        fetch_ids(0)
        wait_ids()
        issue_word(0)
        fetch_ids(1)

        @pl.loop(0, n_chunks)
        def _(ck):
            s3 = lax.rem(ck, 3)
            s2 = lax.rem(ck, 2)
            rowbase = wbase + ck * _CHUNK

            wait_word()
            issue_spk_add(ck)

            @pl.when(ck + 2 < n_chunks)
            def _():
                fetch_ids(ck + 2)

            @pl.when(ck + 1 < n_chunks)
            def _():
                wait_ids()

                @pl.when(ck >= 2)
                def _():
                    pltpu.make_async_copy(
                        obuf.at[0], out_hbm.at[pl.ds(0, _CHUNK)],
                        sem_o).wait()

                issue_word(ck + 1)

            wait_spk_add()

            l0 = lax.rem(rowbase, L)

            @plsc.parallel_loop(0, _CHUNK, unroll=4)
            def _(r):
                lw = l0 + r
                l = jnp.where(lw < L, lw, lw - L)
                for c in range(H // _LANES):
                    sl = pl.ds(c * _LANES, _LANES)
                    plsc.addupdate(obuf.at[s3, r, sl], posbuf[l, sl])

            pltpu.async_copy(obuf.at[s3],
                             out_hbm.at[pl.ds(rowbase, _CHUNK)], sem_o)

        for _ in range(3):
            pltpu.make_async_copy(obuf.at[0], out_hbm.at[pl.ds(0, _CHUNK)],
                                  sem_o).wait()

    return sc_embed


def kernel(input_ids, speaker_ids, word_table, position_table,
           token_type_table, speaker_table):
    B, L = input_ids.shape
    V, H = word_table.shape
    P = speaker_table.shape[0]
    N = B * L
    sc = _build_sc_kernel(N, L, H, P)
    ids = jnp.stack([input_ids.reshape(N).astype(jnp.int32),
                     speaker_ids.reshape(N).astype(jnp.int32)])
    out = sc(ids, word_table, position_table, token_type_table,
             speaker_table)
    return out.reshape(B, L, H)


# speedup vs baseline: 1.2553x; 1.0248x over previous
import functools

import jax
import jax.numpy as jnp
from jax import lax
from jax.experimental import pallas as pl
from jax.experimental.pallas import tpu as pltpu
from jax.experimental.pallas import tpu_sc as plsc

_NC = 2
_NS = 16
_NW = _NC * _NS
_LANES = 16
_CHUNK = 128


def _build_sc_kernel(N, L, H, P, n_pat):
    assert N % (_NW * _CHUNK) == 0
    rows_per_w = N // _NW
    assert rows_per_w % L == 0
    n_chunks = rows_per_w // _CHUNK

    mesh = plsc.VectorSubcoreMesh(core_axis_name="c", subcore_axis_name="s")

    @functools.partial(
        pl.kernel,
        out_type=jax.ShapeDtypeStruct((N, H), jnp.float32),
        mesh=mesh,
        scratch_types=[
            pltpu.VMEM((2, 2, _CHUNK), jnp.int32),
            pltpu.VMEM((3, _CHUNK, H), jnp.float32),
            pltpu.VMEM((n_pat, _CHUNK), jnp.int32),
            pltpu.VMEM((L, H), jnp.float32),
            pltpu.VMEM((H,), jnp.float32),
            pltpu.VMEM_SHARED((L, H), jnp.float32),
            pltpu.VMEM_SHARED((P, H), jnp.float32),
            pltpu.SemaphoreType.DMA,
            pltpu.SemaphoreType.DMA,
            pltpu.SemaphoreType.DMA,
            pltpu.SemaphoreType.DMA,
            pltpu.SemaphoreType.DMA,
        ],
    )
    def sc_embed(ids_hbm, pidx_hbm, word_hbm, pos_hbm, tt_hbm, spk_hbm,
                 out_hbm, idx_v, obuf, pidx_v, posbuf, ttbuf,
                 pos_sp, spk_sp, sem_i, sem_w, sem_s, sem_p, sem_o):
        wid = lax.axis_index("s") * _NC + lax.axis_index("c")
        wbase = wid * rows_per_w

        @pl.when(lax.axis_index("s") == 0)
        def _():
            pltpu.sync_copy(spk_hbm, spk_sp)
            pltpu.sync_copy(pos_hbm.at[pl.ds(0, L)], posbuf)
            pltpu.sync_copy(tt_hbm.at[0], ttbuf)

            @plsc.parallel_loop(0, L, unroll=2)
            def _(r):
                for c in range(H // _LANES):
                    sl = pl.ds(c * _LANES, _LANES)
                    posbuf[r, sl] = posbuf[r, sl] + ttbuf[sl]

            pltpu.sync_copy(posbuf, pos_sp)

        pltpu.sync_copy(pidx_hbm, pidx_v)

        plsc.subcore_barrier()

        def fetch_ids(ck):
            s2 = lax.rem(ck, 2)
            rowbase = wbase + ck * _CHUNK
            pltpu.async_copy(ids_hbm.at[:, pl.ds(rowbase, _CHUNK)],
                             idx_v.at[s2], sem_i)

        def wait_ids():
            pltpu.make_async_copy(ids_hbm.at[:, pl.ds(0, _CHUNK)],
                                  idx_v.at[0], sem_i).wait()

        def issue_word(ck):
            s3 = lax.rem(ck, 3)
            s2 = lax.rem(ck, 2)
            pltpu.async_copy(word_hbm.at[idx_v.at[s2, 0]], obuf.at[s3],
                             sem_w)

        def wait_word():
            pltpu.make_async_copy(word_hbm.at[idx_v.at[0, 0]], obuf.at[0],
                                  sem_w).wait()

        def issue_spk_add(ck):
            s3 = lax.rem(ck, 3)
            s2 = lax.rem(ck, 2)
            pltpu.async_copy(spk_sp.at[idx_v.at[s2, 1]], obuf.at[s3],
                             sem_s, add=True)

        def wait_spk_add():
            pltpu.make_async_copy(spk_sp.at[idx_v.at[0, 1]], obuf.at[0],
                                  sem_s).wait()

        def issue_pos_add(ck):
            s3 = lax.rem(ck, 3)
            p = lax.rem(ck, n_pat)
            pltpu.async_copy(pos_sp.at[pidx_v.at[p]], obuf.at[s3],
                             sem_p, add=True)

        def wait_pos_add():
            pltpu.make_async_copy(pos_sp.at[pidx_v.at[0]], obuf.at[0],
                                  sem_p).wait()

        fetch_ids(0)
        wait_ids()
        issue_word(0)
        fetch_ids(1)

        @pl.loop(0, n_chunks)
        def _(ck):
            s3 = lax.rem(ck, 3)
            rowbase = wbase + ck * _CHUNK

            wait_word()
            issue_spk_add(ck)

            @pl.when(ck + 2 < n_chunks)
            def _():
                fetch_ids(ck + 2)

            @pl.when(ck + 1 < n_chunks)
            def _():
                wait_ids()

                @pl.when(ck >= 2)
                def _():
                    pltpu.make_async_copy(
                        obuf.at[0], out_hbm.at[pl.ds(0, _CHUNK)],
                        sem_o).wait()

                issue_word(ck + 1)

            wait_spk_add()
            issue_pos_add(ck)
            wait_pos_add()

            pltpu.async_copy(obuf.at[s3],
                             out_hbm.at[pl.ds(rowbase, _CHUNK)], sem_o)

        for _ in range(3):
            pltpu.make_async_copy(obuf.at[0], out_hbm.at[pl.ds(0, _CHUNK)],
                                  sem_o).wait()

    return sc_embed


def kernel(input_ids, speaker_ids, word_table, position_table,
           token_type_table, speaker_table):
    B, L = input_ids.shape
    V, H = word_table.shape
    P = speaker_table.shape[0]
    N = B * L
    import math
    n_pat = math.lcm(_CHUNK, L) // _CHUNK
    sc = _build_sc_kernel(N, L, H, P, n_pat)
    ids = jnp.stack([input_ids.reshape(N).astype(jnp.int32),
                     speaker_ids.reshape(N).astype(jnp.int32)])
    pidx = (jnp.arange(n_pat * _CHUNK, dtype=jnp.int32) % L).reshape(
        n_pat, _CHUNK)
    out = sc(ids, pidx, word_table, position_table, token_type_table,
             speaker_table)
    return out.reshape(B, L, H)


# speedup vs baseline: 1.2596x; 1.0035x over previous
import functools

import jax
import jax.numpy as jnp
from jax import lax
from jax.experimental import pallas as pl
from jax.experimental.pallas import tpu as pltpu
from jax.experimental.pallas import tpu_sc as plsc

_NC = 2
_NS = 16
_NW = _NC * _NS
_LANES = 16
_CHUNK = 128


def _build_sc_kernel(N, L, H, P, n_pat):
    assert N % (_NW * _CHUNK) == 0
    rows_per_w = N // _NW
    assert rows_per_w % L == 0
    n_chunks = rows_per_w // _CHUNK

    mesh = plsc.VectorSubcoreMesh(core_axis_name="c", subcore_axis_name="s")

    @functools.partial(
        pl.kernel,
        out_type=jax.ShapeDtypeStruct((N, H), jnp.float32),
        mesh=mesh,
        scratch_types=[
            pltpu.VMEM((2, 2, _CHUNK), jnp.int32),
            pltpu.VMEM((3, _CHUNK, H), jnp.float32),
            pltpu.VMEM((n_pat, _CHUNK), jnp.int32),
            pltpu.VMEM((L, H), jnp.float32),
            pltpu.VMEM((H,), jnp.float32),
            pltpu.VMEM_SHARED((L, H), jnp.float32),
            pltpu.VMEM_SHARED((P, H), jnp.float32),
            pltpu.SemaphoreType.DMA,
            pltpu.SemaphoreType.DMA,
            pltpu.SemaphoreType.DMA,
            pltpu.SemaphoreType.DMA,
            pltpu.SemaphoreType.DMA,
        ],
    )
    def sc_embed(ids_hbm, pidx_hbm, word_hbm, pos_hbm, tt_hbm, spk_hbm,
                 out_hbm, idx_v, obuf, pidx_v, posbuf, ttbuf,
                 pos_sp, spk_sp, sem_i, sem_w, sem_s, sem_p, sem_o):
        wid = lax.axis_index("s") * _NC + lax.axis_index("c")
        wbase = wid * rows_per_w

        @pl.when(lax.axis_index("s") == 0)
        def _():
            pltpu.sync_copy(spk_hbm, spk_sp)
            pltpu.sync_copy(pos_hbm.at[pl.ds(0, L)], posbuf)
            pltpu.sync_copy(tt_hbm.at[0], ttbuf)

            @plsc.parallel_loop(0, L, unroll=2)
            def _(r):
                for c in range(H // _LANES):
                    sl = pl.ds(c * _LANES, _LANES)
                    posbuf[r, sl] = posbuf[r, sl] + ttbuf[sl]

            pltpu.sync_copy(posbuf, pos_sp)

        pltpu.sync_copy(pidx_hbm, pidx_v)

        plsc.subcore_barrier()

        def fetch_ids(ck):
            s2 = lax.rem(ck, 2)
            rowbase = wbase + ck * _CHUNK
            pltpu.async_copy(ids_hbm.at[:, pl.ds(rowbase, _CHUNK)],
                             idx_v.at[s2], sem_i)

        def wait_ids():
            pltpu.make_async_copy(ids_hbm.at[:, pl.ds(0, _CHUNK)],
                                  idx_v.at[0], sem_i).wait()

        def issue_word(ck):
            s3 = lax.rem(ck, 3)
            s2 = lax.rem(ck, 2)
            pltpu.async_copy(word_hbm.at[idx_v.at[s2, 0]], obuf.at[s3],
                             sem_w)

        def wait_word():
            pltpu.make_async_copy(word_hbm.at[idx_v.at[0, 0]], obuf.at[0],
                                  sem_w).wait()

        def issue_spk_add(ck):
            s3 = lax.rem(ck, 3)
            s2 = lax.rem(ck, 2)
            pltpu.async_copy(spk_sp.at[idx_v.at[s2, 1]], obuf.at[s3],
                             sem_s, add=True)

        def wait_spk_add():
            pltpu.make_async_copy(spk_sp.at[idx_v.at[0, 1]], obuf.at[0],
                                  sem_s).wait()

        def issue_pos_add(ck):
            s3 = lax.rem(ck, 3)
            p = lax.rem(ck, n_pat)
            pltpu.async_copy(pos_sp.at[pidx_v.at[p]], obuf.at[s3],
                             sem_p, add=True)

        def wait_pos_add():
            pltpu.make_async_copy(pos_sp.at[pidx_v.at[0]], obuf.at[0],
                                  sem_p).wait()

        def issue_out(ck):
            s3 = lax.rem(ck, 3)
            rowbase = wbase + ck * _CHUNK
            pltpu.async_copy(obuf.at[s3],
                             out_hbm.at[pl.ds(rowbase, _CHUNK)], sem_o)

        fetch_ids(0)
        wait_ids()
        issue_word(0)
        fetch_ids(1)

        @pl.loop(0, n_chunks)
        def _(ck):
            @pl.when(ck >= 1)
            def _():
                wait_pos_add()
                issue_out(ck - 1)

            wait_word()
            issue_spk_add(ck)

            @pl.when(ck + 2 < n_chunks)
            def _():
                fetch_ids(ck + 2)

            @pl.when(ck + 1 < n_chunks)
            def _():
                wait_ids()

                @pl.when(ck >= 2)
                def _():
                    pltpu.make_async_copy(
                        obuf.at[0], out_hbm.at[pl.ds(0, _CHUNK)],
                        sem_o).wait()

                issue_word(ck + 1)

            wait_spk_add()
            issue_pos_add(ck)

        wait_pos_add()
        issue_out(n_chunks - 1)
        for _ in range(3):
            pltpu.make_async_copy(obuf.at[0], out_hbm.at[pl.ds(0, _CHUNK)],
                                  sem_o).wait()

    return sc_embed


def kernel(input_ids, speaker_ids, word_table, position_table,
           token_type_table, speaker_table):
    B, L = input_ids.shape
    V, H = word_table.shape
    P = speaker_table.shape[0]
    N = B * L
    import math
    n_pat = math.lcm(_CHUNK, L) // _CHUNK
    sc = _build_sc_kernel(N, L, H, P, n_pat)
    ids = jnp.stack([input_ids.reshape(N).astype(jnp.int32),
                     speaker_ids.reshape(N).astype(jnp.int32)])
    pidx = (jnp.arange(n_pat * _CHUNK, dtype=jnp.int32) % L).reshape(
        n_pat, _CHUNK)
    out = sc(ids, pidx, word_table, position_table, token_type_table,
             speaker_table)
    return out.reshape(B, L, H)
